# baseline (speedup 1.0000x reference)
import functools
import math

import jax
import jax.numpy as jnp
import numpy as np
from jax import lax
from jax.experimental import pallas as pl
from jax.experimental.pallas import tpu as pltpu
from jax.experimental.pallas import tpu_sc as plsc

FBLK = 256

SC_NC, SC_NS, SC_LANES = 2, 16, 16
SC_NW = SC_NC * SC_NS


@functools.lru_cache(maxsize=None)
def _twiddles(L: int):
    nf_live = L // 2 + 1
    NF = ((nf_live + FBLK - 1) // FBLK) * FBLK
    f = np.arange(NF, dtype=np.int64)
    t = np.arange(L, dtype=np.int64)
    ang = 2.0 * np.pi * ((f[:, None] * t[None, :]) % L).astype(np.float64) / L
    cos = np.cos(ang)
    sin = np.sin(ang)
    live = (f <= L // 2).astype(np.float64)
    cos *= live[:, None]
    sin *= live[:, None]
    w = np.where((f == 0) | (f == L // 2), 1.0, 2.0) / L * live
    icos = (w[:, None] * cos).astype(np.float32)
    isin = (w[:, None] * sin).astype(np.float32)
    fcos = np.ascontiguousarray(cos.T).astype(np.float32)
    fsin = np.ascontiguousarray(sin.T).astype(np.float32)
    return fcos, fsin, icos, isin, NF


def _dot(a, b):
    return jax.lax.dot_general(
        a, b, (((1,), (0,)), ((), ())),
        preferred_element_type=jnp.float32,
        precision=jax.lax.Precision.HIGHEST)


def _dot_t(a, b):
    return jax.lax.dot_general(
        a, b, (((0,), (0,)), ((), ())),
        preferred_element_type=jnp.float32,
        precision=jax.lax.Precision.HIGHEST)


def _corr_body(qf_ref, kf_ref, fcos_ref, fsin_ref, icos_ref, isin_ref, c_ref):
    j = pl.program_id(1)
    q = qf_ref[0]
    k = kf_ref[0]
    fcos = fcos_ref[...]
    fsin = fsin_ref[...]
    qr = _dot_t(q, fcos)
    qs = _dot_t(q, fsin)
    kr = _dot_t(k, fcos)
    ks = _dot_t(k, fsin)
    sr = jnp.sum(qr * kr + qs * ks, axis=0, keepdims=True)
    si = jnp.sum(qr * ks - qs * kr, axis=0, keepdims=True)
    contrib = _dot(sr, icos_ref[...]) - _dot(si, isin_ref[...])

    @pl.when(j == 0)
    def _():
        c_ref[0] = contrib

    @pl.when(j > 0)
    def _():
        c_ref[0] = c_ref[0] + contrib


def _corr_mean(qf, kf, L):
    B, _, HE = qf.shape
    fcos, fsin, icos, isin, NF = _twiddles(L)
    nj = NF // FBLK
    return pl.pallas_call(
        _corr_body,
        grid=(B, nj),
        in_specs=[
            pl.BlockSpec((1, L, HE), lambda b, j: (b, 0, 0)),
            pl.BlockSpec((1, L, HE), lambda b, j: (b, 0, 0)),
            pl.BlockSpec((L, FBLK), lambda b, j: (0, j)),
            pl.BlockSpec((L, FBLK), lambda b, j: (0, j)),
            pl.BlockSpec((FBLK, L), lambda b, j: (j, 0)),
            pl.BlockSpec((FBLK, L), lambda b, j: (j, 0)),
        ],
        out_specs=pl.BlockSpec((1, 1, L), lambda b, j: (b, 0, 0)),
        out_shape=jax.ShapeDtypeStruct((B, 1, L), jnp.float32),
        compiler_params=pltpu.CompilerParams(
            dimension_semantics=("arbitrary", "arbitrary")),
    )(qf, kf, jnp.asarray(fcos), jnp.asarray(fsin),
      jnp.asarray(icos), jnp.asarray(isin))


def _topk_body(B, L, HE, TOPK, c_ref, idx_ref, sw_ref):
    c = c_ref[...].reshape(B, L)
    mc = c * (1.0 / HE)
    gm = jnp.sum(mc, axis=0, keepdims=True) * (1.0 / B)
    lane = lax.broadcasted_iota(jnp.int32, (1, L), 1)
    neg = jnp.float32(-jnp.inf)

    cur = gm
    idxs = []
    for _ in range(TOPK):
        m = jnp.max(cur)
        idx = jnp.min(jnp.where(cur == m, lane, L)).astype(jnp.int32)
        idxs.append(idx)
        cur = jnp.where(lane == idx, neg, cur)

    lane128 = lax.broadcasted_iota(jnp.int32, (1, 128), 1)
    for b in range(B):
        mc_b = mc[b:b + 1, :]
        wrow = jnp.zeros((1, 128), jnp.float32)
        for kk in range(TOPK):
            w_bk = jnp.sum(jnp.where(lane == idxs[kk], mc_b, 0.0))
            wrow = jnp.where(lane128 == kk, w_bk, wrow)
        wrow = jnp.where(lane128 < TOPK, wrow, neg)
        mb = jnp.max(wrow)
        e = jnp.exp(wrow - mb)
        s = jnp.sum(e)
        swrow = e / s
        for kk in range(TOPK):
            sw_bk = jnp.sum(jnp.where(lane128 == kk, swrow, 0.0))
            sw_ref[pl.ds(b * TOPK + kk, 1), :] = jnp.full((1, SC_LANES), sw_bk)

    for kk in range(TOPK):
        s_k = idxs[kk]
        rel = jnp.where(lane >= s_k, lane - s_k, lane + (L - s_k))
        for b in range(B):
            idx_ref[pl.ds(b * TOPK + kk, 1), :] = rel + b * L


def _topk_weights(c3, B, L, HE, TOPK):
    body = functools.partial(_topk_body, B, L, HE, TOPK)
    return pl.pallas_call(
        body,
        grid=(1,),
        in_specs=[pl.BlockSpec((B, 1, L), lambda i: (0, 0, 0))],
        out_specs=[
            pl.BlockSpec((B * TOPK, L), lambda i: (0, 0)),
            pl.BlockSpec((B * TOPK, SC_LANES), lambda i: (0, 0)),
        ],
        out_shape=[
            jax.ShapeDtypeStruct((B * TOPK, L), jnp.int32),
            jax.ShapeDtypeStruct((B * TOPK, SC_LANES), jnp.float32),
        ],
    )(c3)


def _sc_gather_agg(vflat, src_idx, sw16, B, L, HE, TOPK):
    ROWS = B * L
    RW = ROWS // SC_NW
    RT = 16
    NSUB = RW // RT
    mesh = plsc.VectorSubcoreMesh(core_axis_name="c", subcore_axis_name="s")
    scratch = ([pltpu.VMEM((RT,), jnp.int32) for _ in range(TOPK)]
               + [pltpu.VMEM((RT, HE), jnp.float32) for _ in range(TOPK)]
               + [pltpu.VMEM((RT, HE), jnp.float32),
                  pltpu.VMEM((B * TOPK, SC_LANES), jnp.float32),
                  pltpu.SemaphoreType.DMA,
                  pltpu.SemaphoreType.DMA,
                  pltpu.SemaphoreType.DMA])

    @functools.partial(
        pl.kernel,
        out_type=jax.ShapeDtypeStruct((ROWS, HE), jnp.float32),
        mesh=mesh,
        scratch_types=scratch)
    def k(v_hbm, idx_hbm, sw_hbm, out_hbm, *sc):
        idxb = sc[0:TOPK]
        rowb = sc[TOPK:2 * TOPK]
        acc = sc[2 * TOPK]
        sw_all = sc[2 * TOPK + 1]
        sem_i, sem_g, sem_o = sc[2 * TOPK + 2], sc[2 * TOPK + 3], sc[2 * TOPK + 4]

        wid = lax.axis_index("c") * SC_NS + lax.axis_index("s")
        base = wid * RW
        b = base // L
        lbase = base - b * L
        pltpu.sync_copy(sw_hbm, sw_all)

        @pl.loop(0, NSUB)
        def _(st):
            loc = lbase + st * RT
            idx_cps = []
            for kk in range(TOPK):
                idx_cps.append(pltpu.async_copy(
                    idx_hbm.at[b * TOPK + kk, pl.ds(loc, RT)], idxb[kk], sem_i))
            gat_cps = []
            for kk in range(TOPK):
                idx_cps[kk].wait()
                gat_cps.append(pltpu.async_copy(
                    v_hbm.at[idxb[kk]], rowb[kk], sem_g))
            for kk in range(TOPK):
                gat_cps[kk].wait()
                wv = sw_all[b * TOPK + kk]
                rb = rowb[kk]
                if kk == 0:
                    @pl.loop(0, RT)
                    def _(r):
                        @pl.loop(0, HE, step=SC_LANES)
                        def _(cc):
                            acc[r, pl.ds(cc, SC_LANES)] = (
                                wv * rb[r, pl.ds(cc, SC_LANES)])
                else:
                    @pl.loop(0, RT)
                    def _(r):
                        @pl.loop(0, HE, step=SC_LANES)
                        def _(cc):
                            plsc.addupdate(
                                acc.at[r, pl.ds(cc, SC_LANES)],
                                wv * rb[r, pl.ds(cc, SC_LANES)])
            pltpu.async_copy(acc, out_hbm.at[pl.ds(base + st * RT, RT)],
                             sem_o).wait()

    return k(vflat, src_idx, sw16)


def kernel(queries, keys, values, attn_mask):
    B, L, H, E = queries.shape
    HE = H * E
    TOPK = max(1, min(int(1 * math.log(L)), L))

    qf = queries.reshape(B, L, HE)
    kf = keys.reshape(B, L, HE)

    c3 = _corr_mean(qf, kf, L)
    src_idx, sw16 = _topk_weights(c3, B, L, HE, TOPK)

    vflat = values.reshape(B * L, HE)
    out_flat = _sc_gather_agg(vflat, src_idx, sw16, B, L, HE, TOPK)
    return out_flat.reshape(B, L, H, E)


# speedup vs baseline: 1.3059x; 1.3059x over previous
import functools
import math

import jax
import jax.numpy as jnp
import numpy as np
from jax import lax
from jax.experimental import pallas as pl
from jax.experimental.pallas import tpu as pltpu
from jax.experimental.pallas import tpu_sc as plsc

FBLK = 256

SC_NC, SC_NS, SC_LANES = 2, 16, 16
SC_NW = SC_NC * SC_NS


@functools.lru_cache(maxsize=None)
def _twiddles(L: int):
    nf_live = L // 2 + 1
    NF = ((nf_live + FBLK - 1) // FBLK) * FBLK
    f = np.arange(NF, dtype=np.int64)
    t = np.arange(L, dtype=np.int64)
    ang = 2.0 * np.pi * ((f[:, None] * t[None, :]) % L).astype(np.float64) / L
    cos = np.cos(ang)
    sin = np.sin(ang)
    live = (f <= L // 2).astype(np.float64)
    cos *= live[:, None]
    sin *= live[:, None]
    w = np.where((f == 0) | (f == L // 2), 1.0, 2.0) / L * live
    icos = (w[:, None] * cos).astype(np.float32)
    isin = (w[:, None] * sin).astype(np.float32)
    fcos = np.ascontiguousarray(cos.T).astype(np.float32)
    fsin = np.ascontiguousarray(sin.T).astype(np.float32)

    def _split(x):
        hi = x.astype(jnp.bfloat16)
        lo = (x - hi.astype(np.float32)).astype(jnp.bfloat16)
        return hi, lo

    fcos_h, fcos_l = _split(fcos)
    fsin_h, fsin_l = _split(fsin)
    return fcos_h, fcos_l, fsin_h, fsin_l, icos, isin, NF


def _dot(a, b):
    return jax.lax.dot_general(
        a, b, (((1,), (0,)), ((), ())),
        preferred_element_type=jnp.float32,
        precision=jax.lax.Precision.HIGHEST)


def _dot_bf(a, b):
    return jax.lax.dot_general(
        a, b, (((1,), (0,)), ((), ())),
        preferred_element_type=jnp.float32)


def _dot3(ah, al, th, tl):
    return _dot_bf(ah, th) + _dot_bf(ah, tl) + _dot_bf(al, th)


def _corr_body(qfT_ref, kfT_ref, fch_ref, fcl_ref, fsh_ref, fsl_ref,
               icos_ref, isin_ref, c_ref):
    j = pl.program_id(1)

    def split(x):
        hi = x.astype(jnp.bfloat16)
        lo = (x - hi.astype(jnp.float32)).astype(jnp.bfloat16)
        return hi, lo

    qh, ql = split(qfT_ref[0])
    kh, kl = split(kfT_ref[0])
    fch, fcl = fch_ref[...], fcl_ref[...]
    fsh, fsl = fsh_ref[...], fsl_ref[...]
    qr = _dot3(qh, ql, fch, fcl)
    qs = _dot3(qh, ql, fsh, fsl)
    kr = _dot3(kh, kl, fch, fcl)
    ks = _dot3(kh, kl, fsh, fsl)
    sr = jnp.sum(qr * kr + qs * ks, axis=0, keepdims=True)
    si = jnp.sum(qr * ks - qs * kr, axis=0, keepdims=True)
    contrib = _dot(sr, icos_ref[...]) - _dot(si, isin_ref[...])

    @pl.when(j == 0)
    def _():
        c_ref[0] = contrib

    @pl.when(j > 0)
    def _():
        c_ref[0] = c_ref[0] + contrib


def _corr_mean(qfT, kfT, L):
    B, HE, _ = qfT.shape
    fch, fcl, fsh, fsl, icos, isin, NF = _twiddles(L)
    nj = NF // FBLK
    return pl.pallas_call(
        _corr_body,
        grid=(B, nj),
        in_specs=[
            pl.BlockSpec((1, HE, L), lambda b, j: (b, 0, 0)),
            pl.BlockSpec((1, HE, L), lambda b, j: (b, 0, 0)),
            pl.BlockSpec((L, FBLK), lambda b, j: (0, j)),
            pl.BlockSpec((L, FBLK), lambda b, j: (0, j)),
            pl.BlockSpec((L, FBLK), lambda b, j: (0, j)),
            pl.BlockSpec((L, FBLK), lambda b, j: (0, j)),
            pl.BlockSpec((FBLK, L), lambda b, j: (j, 0)),
            pl.BlockSpec((FBLK, L), lambda b, j: (j, 0)),
        ],
        out_specs=pl.BlockSpec((1, 1, L), lambda b, j: (b, 0, 0)),
        out_shape=jax.ShapeDtypeStruct((B, 1, L), jnp.float32),
        compiler_params=pltpu.CompilerParams(
            dimension_semantics=("arbitrary", "arbitrary")),
    )(qfT, kfT, jnp.asarray(fch), jnp.asarray(fcl),
      jnp.asarray(fsh), jnp.asarray(fsl),
      jnp.asarray(icos), jnp.asarray(isin))


def _topk_body(B, L, HE, TOPK, c_ref, idx_ref, sw_ref):
    c = c_ref[...].reshape(B, L)
    mc = c * (1.0 / HE)
    gm = jnp.sum(mc, axis=0, keepdims=True) * (1.0 / B)
    lane = lax.broadcasted_iota(jnp.int32, (1, L), 1)
    neg = jnp.float32(-jnp.inf)

    cur = gm
    idxs = []
    for _ in range(TOPK):
        m = jnp.max(cur)
        idx = jnp.min(jnp.where(cur == m, lane, L)).astype(jnp.int32)
        idxs.append(idx)
        cur = jnp.where(lane == idx, neg, cur)

    lane128 = lax.broadcasted_iota(jnp.int32, (1, 128), 1)
    for b in range(B):
        mc_b = mc[b:b + 1, :]
        wrow = jnp.zeros((1, 128), jnp.float32)
        for kk in range(TOPK):
            w_bk = jnp.sum(jnp.where(lane == idxs[kk], mc_b, 0.0))
            wrow = jnp.where(lane128 == kk, w_bk, wrow)
        wrow = jnp.where(lane128 < TOPK, wrow, neg)
        mb = jnp.max(wrow)
        e = jnp.exp(wrow - mb)
        s = jnp.sum(e)
        swrow = e / s
        for kk in range(TOPK):
            sw_bk = jnp.sum(jnp.where(lane128 == kk, swrow, 0.0))
            sw_ref[pl.ds(b * TOPK + kk, 1), :] = jnp.full((1, SC_LANES), sw_bk)

    for kk in range(TOPK):
        s_k = idxs[kk]
        rel = jnp.where(lane >= s_k, lane - s_k, lane + (L - s_k))
        for b in range(B):
            idx_ref[pl.ds(b * TOPK + kk, 1), :] = rel + b * L


def _topk_weights(c3, B, L, HE, TOPK):
    body = functools.partial(_topk_body, B, L, HE, TOPK)
    return pl.pallas_call(
        body,
        grid=(1,),
        in_specs=[pl.BlockSpec((B, 1, L), lambda i: (0, 0, 0))],
        out_specs=[
            pl.BlockSpec((B * TOPK, L), lambda i: (0, 0)),
            pl.BlockSpec((B * TOPK, SC_LANES), lambda i: (0, 0)),
        ],
        out_shape=[
            jax.ShapeDtypeStruct((B * TOPK, L), jnp.int32),
            jax.ShapeDtypeStruct((B * TOPK, SC_LANES), jnp.float32),
        ],
    )(c3)


def _sc_gather_agg(vflat, src_idx, sw16, B, L, HE, TOPK):
    ROWS = B * L
    RW = ROWS // SC_NW
    RT = 16
    NSUB = RW // RT
    mesh = plsc.VectorSubcoreMesh(core_axis_name="c", subcore_axis_name="s")
    scratch = ([pltpu.VMEM((RT,), jnp.int32) for _ in range(TOPK)]
               + [pltpu.VMEM((RT, HE), jnp.float32) for _ in range(TOPK)]
               + [pltpu.VMEM((RT, HE), jnp.float32),
                  pltpu.VMEM((B * TOPK, SC_LANES), jnp.float32),
                  pltpu.SemaphoreType.DMA,
                  pltpu.SemaphoreType.DMA,
                  pltpu.SemaphoreType.DMA])

    @functools.partial(
        pl.kernel,
        out_type=jax.ShapeDtypeStruct((ROWS, HE), jnp.float32),
        mesh=mesh,
        scratch_types=scratch)
    def k(v_hbm, idx_hbm, sw_hbm, out_hbm, *sc):
        idxb = sc[0:TOPK]
        rowb = sc[TOPK:2 * TOPK]
        acc = sc[2 * TOPK]
        sw_all = sc[2 * TOPK + 1]
        sem_i, sem_g, sem_o = sc[2 * TOPK + 2], sc[2 * TOPK + 3], sc[2 * TOPK + 4]

        wid = lax.axis_index("c") * SC_NS + lax.axis_index("s")
        base = wid * RW
        b = base // L
        lbase = base - b * L
        pltpu.sync_copy(sw_hbm, sw_all)

        @pl.loop(0, NSUB)
        def _(st):
            loc = lbase + st * RT
            idx_cps = []
            for kk in range(TOPK):
                idx_cps.append(pltpu.async_copy(
                    idx_hbm.at[b * TOPK + kk, pl.ds(loc, RT)], idxb[kk], sem_i))
            gat_cps = []
            for kk in range(TOPK):
                idx_cps[kk].wait()
                gat_cps.append(pltpu.async_copy(
                    v_hbm.at[idxb[kk]], rowb[kk], sem_g))
            for kk in range(TOPK):
                gat_cps[kk].wait()
                wv = sw_all[b * TOPK + kk]
                rb = rowb[kk]
                if kk == 0:
                    @pl.loop(0, RT)
                    def _(r):
                        @pl.loop(0, HE, step=SC_LANES)
                        def _(cc):
                            acc[r, pl.ds(cc, SC_LANES)] = (
                                wv * rb[r, pl.ds(cc, SC_LANES)])
                else:
                    @pl.loop(0, RT)
                    def _(r):
                        @pl.loop(0, HE, step=SC_LANES)
                        def _(cc):
                            plsc.addupdate(
                                acc.at[r, pl.ds(cc, SC_LANES)],
                                wv * rb[r, pl.ds(cc, SC_LANES)])
            pltpu.async_copy(acc, out_hbm.at[pl.ds(base + st * RT, RT)],
                             sem_o).wait()

    return k(vflat, src_idx, sw16)


def kernel(queries, keys, values, attn_mask):
    B, L, H, E = queries.shape
    HE = H * E
    TOPK = max(1, min(int(1 * math.log(L)), L))

    qfT = jnp.transpose(queries.reshape(B, L, HE), (0, 2, 1))
    kfT = jnp.transpose(keys.reshape(B, L, HE), (0, 2, 1))

    c3 = _corr_mean(qfT, kfT, L)
    src_idx, sw16 = _topk_weights(c3, B, L, HE, TOPK)

    vflat = values.reshape(B * L, HE)
    out_flat = _sc_gather_agg(vflat, src_idx, sw16, B, L, HE, TOPK)
    return out_flat.reshape(B, L, H, E)


# speedup vs baseline: 1.7830x; 1.3653x over previous
import functools
import math

import jax
import jax.numpy as jnp
import numpy as np
from jax import lax
from jax.experimental import pallas as pl
from jax.experimental.pallas import tpu as pltpu
from jax.experimental.pallas import tpu_sc as plsc

FBLK = 256

SC_NC, SC_NS, SC_LANES = 2, 16, 16
SC_NW = SC_NC * SC_NS


@functools.lru_cache(maxsize=None)
def _twiddles(L: int):
    nf_live = L // 2 + 1
    NF = ((nf_live + FBLK - 1) // FBLK) * FBLK
    f = np.arange(NF, dtype=np.int64)
    t = np.arange(L, dtype=np.int64)
    ang = 2.0 * np.pi * ((f[:, None] * t[None, :]) % L).astype(np.float64) / L
    cos = np.cos(ang)
    sin = np.sin(ang)
    live = (f <= L // 2).astype(np.float64)
    cos *= live[:, None]
    sin *= live[:, None]
    w = np.where((f == 0) | (f == L // 2), 1.0, 2.0) / L * live
    icos = (w[:, None] * cos).astype(np.float32)
    isin = (w[:, None] * sin).astype(np.float32)
    fcos = np.ascontiguousarray(cos.T).astype(np.float32)
    fsin = np.ascontiguousarray(sin.T).astype(np.float32)

    def _split(x):
        hi = x.astype(jnp.bfloat16)
        lo = (x - hi.astype(np.float32)).astype(jnp.bfloat16)
        return hi, lo

    fcos_h, fcos_l = _split(fcos)
    fsin_h, fsin_l = _split(fsin)
    return fcos_h, fcos_l, fsin_h, fsin_l, icos, isin, NF


def _dot(a, b):
    return jax.lax.dot_general(
        a, b, (((1,), (0,)), ((), ())),
        preferred_element_type=jnp.float32,
        precision=jax.lax.Precision.HIGHEST)


def _dot_bf(a, b):
    return jax.lax.dot_general(
        a, b, (((1,), (0,)), ((), ())),
        preferred_element_type=jnp.float32)


def _dot3(ah, al, th, tl):
    return _dot_bf(ah, th) + _dot_bf(ah, tl) + _dot_bf(al, th)


def _corr_body(qfT_ref, kfT_ref, fch_ref, fcl_ref, fsh_ref, fsl_ref,
               icos_ref, isin_ref, c_ref):
    j = pl.program_id(1)

    def split(x):
        hi = x.astype(jnp.bfloat16)
        lo = (x - hi.astype(jnp.float32)).astype(jnp.bfloat16)
        return hi, lo

    qh, ql = split(qfT_ref[0])
    kh, kl = split(kfT_ref[0])
    fch, fcl = fch_ref[...], fcl_ref[...]
    fsh, fsl = fsh_ref[...], fsl_ref[...]
    qr = _dot3(qh, ql, fch, fcl)
    qs = _dot3(qh, ql, fsh, fsl)
    kr = _dot3(kh, kl, fch, fcl)
    ks = _dot3(kh, kl, fsh, fsl)
    sr = jnp.sum(qr * kr + qs * ks, axis=0, keepdims=True)
    si = jnp.sum(qr * ks - qs * kr, axis=0, keepdims=True)
    contrib = _dot(sr, icos_ref[...]) - _dot(si, isin_ref[...])

    @pl.when(j == 0)
    def _():
        c_ref[0] = contrib

    @pl.when(j > 0)
    def _():
        c_ref[0] = c_ref[0] + contrib


def _corr_mean(qfT, kfT, L):
    B, HE, _ = qfT.shape
    fch, fcl, fsh, fsl, icos, isin, NF = _twiddles(L)
    nj = NF // FBLK
    return pl.pallas_call(
        _corr_body,
        grid=(B, nj),
        in_specs=[
            pl.BlockSpec((1, HE, L), lambda b, j: (b, 0, 0)),
            pl.BlockSpec((1, HE, L), lambda b, j: (b, 0, 0)),
            pl.BlockSpec((L, FBLK), lambda b, j: (0, j)),
            pl.BlockSpec((L, FBLK), lambda b, j: (0, j)),
            pl.BlockSpec((L, FBLK), lambda b, j: (0, j)),
            pl.BlockSpec((L, FBLK), lambda b, j: (0, j)),
            pl.BlockSpec((FBLK, L), lambda b, j: (j, 0)),
            pl.BlockSpec((FBLK, L), lambda b, j: (j, 0)),
        ],
        out_specs=pl.BlockSpec((1, 1, L), lambda b, j: (b, 0, 0)),
        out_shape=jax.ShapeDtypeStruct((B, 1, L), jnp.float32),
        compiler_params=pltpu.CompilerParams(
            dimension_semantics=("arbitrary", "arbitrary")),
    )(qfT, kfT, jnp.asarray(fch), jnp.asarray(fcl),
      jnp.asarray(fsh), jnp.asarray(fsl),
      jnp.asarray(icos), jnp.asarray(isin))


KPAD = 8


def _topk_body(B, L, HE, TOPK, c_ref, idx_ref, sw_ref):
    c = c_ref[...].reshape(B, L)
    mc = c * (1.0 / HE)
    gm = jnp.sum(mc, axis=0, keepdims=True) * (1.0 / B)
    lane = lax.broadcasted_iota(jnp.int32, (1, L), 1)
    neg = jnp.float32(-jnp.inf)

    cur = gm
    idxs = []
    for _ in range(TOPK):
        m = jnp.max(cur)
        idx = jnp.min(jnp.where(cur == m, lane, L)).astype(jnp.int32)
        idxs.append(idx)
        cur = jnp.where(lane == idx, neg, cur)

    lane128 = lax.broadcasted_iota(jnp.int32, (1, 128), 1)
    for b in range(B):
        mc_b = mc[b:b + 1, :]
        wrow = jnp.zeros((1, 128), jnp.float32)
        for kk in range(TOPK):
            w_bk = jnp.sum(jnp.where(lane == idxs[kk], mc_b, 0.0))
            wrow = jnp.where(lane128 == kk, w_bk, wrow)
        wrow = jnp.where(lane128 < TOPK, wrow, neg)
        mb = jnp.max(wrow)
        e = jnp.exp(wrow - mb)
        s = jnp.sum(e)
        swrow = e / s
        for kk in range(TOPK):
            sw_bk = jnp.sum(jnp.where(lane128 == kk, swrow, 0.0))
            sw_ref[pl.ds(b * KPAD + kk, 1), :] = jnp.full((1, SC_LANES), sw_bk)

    for kk in range(TOPK):
        s_k = idxs[kk]
        rel = jnp.where(lane >= s_k, lane - s_k, lane + (L - s_k))
        for b in range(B):
            idx_ref[pl.ds(b * KPAD + kk, 1), :] = rel + b * L


def _topk_weights(c3, B, L, HE, TOPK):
    body = functools.partial(_topk_body, B, L, HE, TOPK)
    return pl.pallas_call(
        body,
        grid=(1,),
        in_specs=[pl.BlockSpec((B, 1, L), lambda i: (0, 0, 0))],
        out_specs=[
            pl.BlockSpec((B * KPAD, L), lambda i: (0, 0)),
            pl.BlockSpec((B * KPAD, SC_LANES), lambda i: (0, 0)),
        ],
        out_shape=[
            jax.ShapeDtypeStruct((B * KPAD, L), jnp.int32),
            jax.ShapeDtypeStruct((B * KPAD, SC_LANES), jnp.float32),
        ],
    )(c3)


def _sc_gather_agg(vflat, src_idx, sw16, B, L, HE, TOPK):
    ROWS = B * L
    RW = ROWS // SC_NW
    RT = 8
    NSUB = RW // RT
    CC_UNROLL = 3
    mesh = plsc.VectorSubcoreMesh(core_axis_name="c", subcore_axis_name="s")
    scratch = ([pltpu.VMEM((RT, HE), jnp.float32)
                for _ in range(2 * TOPK)]
               + [pltpu.VMEM((RT, HE), jnp.float32),
                  pltpu.VMEM((RT, HE), jnp.float32),
                  pltpu.VMEM((KPAD, RW), jnp.int32),
                  pltpu.VMEM((B * KPAD, SC_LANES), jnp.float32),
                  pltpu.SemaphoreType.DMA,
                  pltpu.SemaphoreType.DMA])

    @functools.partial(
        pl.kernel,
        out_type=jax.ShapeDtypeStruct((ROWS, HE), jnp.float32),
        mesh=mesh,
        scratch_types=scratch)
    def k(v_hbm, idx_hbm, sw_hbm, out_hbm, *sc):
        rowb = [sc[0:TOPK], sc[TOPK:2 * TOPK]]
        acc = [sc[2 * TOPK], sc[2 * TOPK + 1]]
        idx_all = sc[2 * TOPK + 2]
        sw_all = sc[2 * TOPK + 3]
        sem_g, sem_o = sc[2 * TOPK + 4], sc[2 * TOPK + 5]

        wid = lax.axis_index("c") * SC_NS + lax.axis_index("s")
        base = wid * RW
        b = base // L
        lbase = base - b * L
        pltpu.sync_copy(sw_hbm, sw_all)
        pltpu.sync_copy(idx_hbm.at[pl.ds(b * KPAD, KPAD), pl.ds(lbase, RW)],
                        idx_all)
        wv = [sw_all[b * KPAD + kk] for kk in range(TOPK)]

        gat = [None, None]
        out_cp = [None, None]

        def fire_gat(g):
            s = g & 1
            gat[s] = [pltpu.async_copy(
                v_hbm.at[idx_all.at[kk, pl.ds(g * RT, RT)]],
                rowb[s][kk], sem_g) for kk in range(TOPK)]

        fire_gat(0)
        for g in range(NSUB):
            s = g & 1
            if g + 1 < NSUB:
                fire_gat(g + 1)
            for cp in gat[s]:
                cp.wait()
            if out_cp[s] is not None:
                out_cp[s].wait()
            rbs = rowb[s]
            accs = acc[s]

            @pl.loop(0, RT)
            def _(r):
                @pl.loop(0, HE, step=SC_LANES * CC_UNROLL)
                def _(cc):
                    for u in range(CC_UNROLL):
                        sl = (r, pl.ds(cc + u * SC_LANES, SC_LANES))
                        val = wv[0] * rbs[0][sl]
                        for kk in range(1, TOPK):
                            val = val + wv[kk] * rbs[kk][sl]
                        accs[sl] = val

            out_cp[s] = pltpu.async_copy(
                accs, out_hbm.at[pl.ds(base + g * RT, RT)], sem_o)
        for s in range(2):
            if out_cp[s] is not None:
                out_cp[s].wait()

    return k(vflat, src_idx, sw16)


def kernel(queries, keys, values, attn_mask):
    B, L, H, E = queries.shape
    HE = H * E
    TOPK = max(1, min(int(1 * math.log(L)), L))

    qfT = jnp.transpose(queries.reshape(B, L, HE), (0, 2, 1))
    kfT = jnp.transpose(keys.reshape(B, L, HE), (0, 2, 1))

    c3 = _corr_mean(qfT, kfT, L)
    src_idx, sw16 = _topk_weights(c3, B, L, HE, TOPK)

    vflat = values.reshape(B * L, HE)
    out_flat = _sc_gather_agg(vflat, src_idx, sw16, B, L, HE, TOPK)
    return out_flat.reshape(B, L, H, E)
